# baseline (speedup 1.0000x reference)
import jax
import jax.numpy as jnp
from jax import lax
from jax.experimental import pallas as pl
from jax.experimental.pallas import tpu as pltpu

_N, _R, _WM, _HM = 512, 64, 6, 6
_E, _MX, _MY = 16, 32, 24
_CIN = 10 + _E
_C1, _C2, _C3 = 32, 64, 64
_OUT = 64

_XG, _YG = 40, 32
_BUF = _XG * _YG
_ROW0 = 4 * _YG
_NROW = _MX * _YG
_WIN = _WM * _YG + 8


def _conv_taps(bf, wg_ref, k, h, cout):
    base = _ROW0 - _YG * h - h
    g = jnp.concatenate([bf[base + ty:base + ty + 1152, :] for ty in range(k)],
                        axis=1)
    acc = jnp.zeros((_NROW, cout), jnp.float32)
    for tx in range(k):
        acc = acc + jnp.dot(g[_YG * tx:_YG * tx + _NROW, :], wg_ref[tx],
                            preferred_element_type=jnp.float32)
    return acc


def _kernel(pos_smem, rooms_ref, emb_ref, w0_ref, b0_ref, w1_ref, b1_ref,
            w2_ref, b2_ref, rw0_ref, rb0_ref, rw1_ref, rb1_ref, rw2_ref,
            rb2_ref, bg_ref, ym_ref, out_ref, m_ref, m2_ref, m3_ref, m4_ref, m5_ref, m6_ref, m7_ref, m8_ref, f_ref, y_ref,
            vals_ref):
    i = pl.program_id(0)

    @pl.when(i == 0)
    def _build():
        p9 = rooms_ref[...]
        mask = p9[:, :, :, 0:1]
        pe = mask * emb_ref[...][:, None, None, :]
        patch = jnp.concatenate(
            [p9, jnp.zeros((_R, _WM, _HM, 1), jnp.float32), pe], axis=-1)
        pw = jnp.concatenate(
            [patch, jnp.zeros((_R, _WM, _YG - _HM, _CIN), jnp.float32)],
            axis=2).reshape(_R, _WM * _YG, _CIN)
        vals_ref[...] = jnp.concatenate(
            [pw, jnp.zeros((_R, 8, _CIN), jnp.float32)], axis=1)

    m_ref[...] = bg_ref[...]
    m2_ref[...] = jnp.zeros((_BUF, _CIN), jnp.float32)
    m3_ref[...] = jnp.zeros((_BUF, _CIN), jnp.float32)
    m4_ref[...] = jnp.zeros((_BUF, _CIN), jnp.float32)
    m5_ref[...] = jnp.zeros((_BUF, _CIN), jnp.float32)
    m6_ref[...] = jnp.zeros((_BUF, _CIN), jnp.float32)
    m7_ref[...] = jnp.zeros((_BUF, _CIN), jnp.float32)
    m8_ref[...] = jnp.zeros((_BUF, _CIN), jnp.float32)
    mrefs = (m_ref, m2_ref, m3_ref, m4_ref, m5_ref, m6_ref, m7_ref, m8_ref)

    def scatter_body(r, carry):
        for l in range(8):
            rl = 8 * r + l
            px = pos_smem[(i * _R + rl) * 2]
            py = pos_smem[(i * _R + rl) * 2 + 1]
            base = (px + 4) * _YG + py + 2
            mr = mrefs[l]
            cur = mr[pl.ds(base, _WM * _YG), :]
            mr[pl.ds(base, _WM * _YG), :] = cur + vals_ref[rl, 0:_WM * _YG, :]
        return carry

    lax.fori_loop(0, _R // 8, scatter_body, 0)

    ym = ym_ref[...]
    bf = ((m_ref[...] + m2_ref[...]) + (m3_ref[...] + m4_ref[...])) + (
        (m5_ref[...] + m6_ref[...]) + (m7_ref[...] + m8_ref[...]))
    a = _conv_taps(bf, w0_ref, 5, 2, _C1)
    a = jnp.maximum(a + b0_ref[...], 0.0) * ym
    bf = jnp.concatenate(
        [jnp.zeros((_ROW0, _C1), jnp.float32), a,
         jnp.zeros((_ROW0, _C1), jnp.float32)], axis=0)
    a = _conv_taps(bf, w1_ref, 3, 1, _C2)
    a = jnp.maximum(a + b1_ref[...], 0.0) * ym
    bf = jnp.concatenate(
        [jnp.zeros((_ROW0, _C2), jnp.float32), a,
         jnp.zeros((_ROW0, _C2), jnp.float32)], axis=0)
    a = _conv_taps(bf, w2_ref, 3, 1, _C3)
    a = jnp.maximum(a + b2_ref[...], 0.0)
    f_ref[0:_NROW, :] = a
    f_ref[_NROW:, :] = jnp.zeros((_WIN + 8, _C3), jnp.float32)

    def decode_body(r, carry):
        for l in range(8):
            rl = r + l * (_R // 8)
            px = pos_smem[(i * _R + rl) * 2]
            py = pos_smem[(i * _R + rl) * 2 + 1]
            base = px * _YG + py + 2
            win = f_ref[pl.ds(base, _WM * _YG), :]
            w = vals_ref[rl, 0:_WM * _YG, 0:1]
            row = jnp.sum(win * w, axis=0)
            y_ref[pl.ds(rl, 1), :] = row[None, :]
        return carry

    lax.fori_loop(0, _R // 8, decode_body, 0)

    y = y_ref[...]
    h = jnp.maximum(jnp.dot(y, rw0_ref[...],
                            preferred_element_type=jnp.float32) + rb0_ref[...], 0.0)
    h = jnp.maximum(jnp.dot(h, rw1_ref[...],
                            preferred_element_type=jnp.float32) + rb1_ref[...], 0.0)
    o = jnp.dot(h, rw2_ref[...],
                preferred_element_type=jnp.float32) + rb2_ref[...]
    out_ref[0] = o


def kernel(room_positions, rooms, emb, w0, b0, w1, b1, w2, b2,
           rw0, rb0, rw1, rb1, rw2, rb2):
    n = room_positions.shape[0]
    pos_flat = room_positions.astype(jnp.int32).reshape(-1)
    rooms_t = rooms.transpose(0, 2, 3, 1)
    w0t = w0.transpose(2, 3, 1, 0).reshape(5, 5 * _CIN, _C1)
    w1t = w1.transpose(2, 3, 1, 0).reshape(3, 3 * _C1, _C2)
    w2t = w2.transpose(2, 3, 1, 0).reshape(3, 3 * _C2, _C3)
    rw0t, rw1t, rw2t = rw0.T, rw1.T, rw2.T
    b0r, b1r, b2r = b0[None, :], b1[None, :], b2[None, :]
    rb0r, rb1r, rb2r = rb0[None, :], rb1[None, :], rb2[None, :]

    rows = jnp.arange(_BUF, dtype=jnp.int32)
    ry = rows % _YG
    interior = (rows >= _ROW0) & (rows < _ROW0 + _NROW) & (ry >= 2) & (ry < 2 + _MY)
    lane = jnp.arange(_CIN, dtype=jnp.int32)
    bg = (interior[:, None] & (lane[None, :] == 9)).astype(jnp.float32)
    ry_out = jnp.arange(_NROW, dtype=jnp.int32) % _YG
    ym = ((ry_out >= 2) & (ry_out < 2 + _MY)).astype(jnp.float32)[:, None]

    specs = [
        pl.BlockSpec((_R, _WM, _HM, 9), lambda i, p: (0, 0, 0, 0)),
        pl.BlockSpec((_R, _E), lambda i, p: (0, 0)),
        pl.BlockSpec((5, 5 * _CIN, _C1), lambda i, p: (0, 0, 0)),
        pl.BlockSpec((1, _C1), lambda i, p: (0, 0)),
        pl.BlockSpec((3, 3 * _C1, _C2), lambda i, p: (0, 0, 0)),
        pl.BlockSpec((1, _C2), lambda i, p: (0, 0)),
        pl.BlockSpec((3, 3 * _C2, _C3), lambda i, p: (0, 0, 0)),
        pl.BlockSpec((1, _C3), lambda i, p: (0, 0)),
        pl.BlockSpec((_C3, _OUT), lambda i, p: (0, 0)),
        pl.BlockSpec((1, _OUT), lambda i, p: (0, 0)),
        pl.BlockSpec((_OUT, _OUT), lambda i, p: (0, 0)),
        pl.BlockSpec((1, _OUT), lambda i, p: (0, 0)),
        pl.BlockSpec((_OUT, _OUT), lambda i, p: (0, 0)),
        pl.BlockSpec((1, _OUT), lambda i, p: (0, 0)),
        pl.BlockSpec((_BUF, _CIN), lambda i, p: (0, 0)),
        pl.BlockSpec((_NROW, 1), lambda i, p: (0, 0)),
    ]

    grid_spec = pltpu.PrefetchScalarGridSpec(
        num_scalar_prefetch=1,
        grid=(n,),
        in_specs=specs,
        out_specs=pl.BlockSpec((1, _R, _OUT), lambda i, p: (i, 0, 0)),
        scratch_shapes=[
            pltpu.VMEM((_BUF, _CIN), jnp.float32),
            pltpu.VMEM((_BUF, _CIN), jnp.float32),
            pltpu.VMEM((_BUF, _CIN), jnp.float32),
            pltpu.VMEM((_BUF, _CIN), jnp.float32),
            pltpu.VMEM((_BUF, _CIN), jnp.float32),
            pltpu.VMEM((_BUF, _CIN), jnp.float32),
            pltpu.VMEM((_BUF, _CIN), jnp.float32),
            pltpu.VMEM((_BUF, _CIN), jnp.float32),
            pltpu.VMEM((_NROW + _WIN + 8, _C3), jnp.float32),
            pltpu.VMEM((_R, _C3), jnp.float32),
            pltpu.VMEM((_R, _WIN, _CIN), jnp.float32),
        ],
    )

    return pl.pallas_call(
        _kernel,
        out_shape=jax.ShapeDtypeStruct((n, _R, _OUT), jnp.float32),
        grid_spec=grid_spec,
        compiler_params=pltpu.CompilerParams(
            dimension_semantics=("arbitrary",),
        ),
        name="main_network_fused",
    )(pos_flat, rooms_t, emb, w0t, b0r, w1t, b1r, w2t, b2r,
      rw0t, rb0r, rw1t, rb1r, rw2t, rb2r, bg, ym)


# speedup vs baseline: 1.0877x; 1.0877x over previous
import jax
import jax.numpy as jnp
from jax import lax
from jax.experimental import pallas as pl
from jax.experimental.pallas import tpu as pltpu

_N, _R, _WM, _HM = 512, 64, 6, 6
_E, _MX, _MY = 16, 32, 24
_CIN = 10 + _E
_C1, _C2, _C3 = 32, 64, 64
_OUT = 64

_XG, _YG = 40, 32
_BUF = _XG * _YG
_ROW0 = 4 * _YG
_NROW = _MX * _YG
_WIN = _WM * _YG + 8


def _conv_taps(bf, wg_ref, k, h, cout):
    base = _ROW0 - _YG * h - h
    g = jnp.concatenate([bf[base + ty:base + ty + 1152, :] for ty in range(k)],
                        axis=1)
    acc = jnp.zeros((_NROW, cout), jnp.float32)
    for tx in range(k):
        acc = acc + jnp.dot(g[_YG * tx:_YG * tx + _NROW, :], wg_ref[tx],
                            preferred_element_type=jnp.float32)
    return acc


def _kernel(pos_smem, rooms_ref, emb_ref, w0_ref, b0_ref, w1_ref, b1_ref,
            w2_ref, b2_ref, rw0_ref, rb0_ref, rw1_ref, rb1_ref, rw2_ref,
            rb2_ref, bg_ref, ym_ref, out_ref, m_ref, m2_ref, m3_ref, m4_ref, f_ref, y_ref,
            vals_ref):
    i = pl.program_id(0)

    @pl.when(i == 0)
    def _build():
        p9 = rooms_ref[...]
        mask = p9[:, :, :, 0:1]
        pe = mask * emb_ref[...][:, None, None, :]
        patch = jnp.concatenate(
            [p9, jnp.zeros((_R, _WM, _HM, 1), jnp.float32), pe], axis=-1)
        pw = jnp.concatenate(
            [patch, jnp.zeros((_R, _WM, _YG - _HM, _CIN), jnp.float32)],
            axis=2).reshape(_R, _WM * _YG, _CIN)
        vals_ref[...] = jnp.concatenate(
            [pw, jnp.zeros((_R, 8, _CIN), jnp.float32)], axis=1)

    m_ref[...] = bg_ref[...]
    m2_ref[...] = jnp.zeros((_BUF, _CIN), jnp.float32)
    m3_ref[...] = jnp.zeros((_BUF, _CIN), jnp.float32)
    m4_ref[...] = jnp.zeros((_BUF, _CIN), jnp.float32)
    mrefs = (m_ref, m2_ref, m3_ref, m4_ref)

    def scatter_body(r, carry):
        for l in range(4):
            rl = 4 * r + l
            px = pos_smem[(i * _R + rl) * 2]
            py = pos_smem[(i * _R + rl) * 2 + 1]
            base = (px + 4) * _YG + py + 2
            mr = mrefs[l]
            cur = mr[pl.ds(base, _WM * _YG), :]
            mr[pl.ds(base, _WM * _YG), :] = cur + vals_ref[rl, 0:_WM * _YG, :]
        return carry

    lax.fori_loop(0, _R // 4, scatter_body, 0)

    ym = ym_ref[...]
    bf = (m_ref[...] + m2_ref[...]) + (m3_ref[...] + m4_ref[...])
    a = _conv_taps(bf, w0_ref, 5, 2, _C1)
    a = jnp.maximum(a + b0_ref[...], 0.0) * ym
    bf = jnp.concatenate(
        [jnp.zeros((_ROW0, _C1), jnp.float32), a,
         jnp.zeros((_ROW0, _C1), jnp.float32)], axis=0)
    a = _conv_taps(bf, w1_ref, 3, 1, _C2)
    a = jnp.maximum(a + b1_ref[...], 0.0) * ym
    bf = jnp.concatenate(
        [jnp.zeros((_ROW0, _C2), jnp.float32), a,
         jnp.zeros((_ROW0, _C2), jnp.float32)], axis=0)
    a = _conv_taps(bf, w2_ref, 3, 1, _C3)
    a = jnp.maximum(a + b2_ref[...], 0.0)
    f_ref[0:_NROW, :] = a
    f_ref[_NROW:, :] = jnp.zeros((_WIN + 8, _C3), jnp.float32)

    def decode_body(r, carry):
        for l in range(16):
            rl = r + l * (_R // 16)
            px = pos_smem[(i * _R + rl) * 2]
            py = pos_smem[(i * _R + rl) * 2 + 1]
            base = px * _YG + py + 2
            win = f_ref[pl.ds(base, _WM * _YG), :]
            w = vals_ref[rl, 0:_WM * _YG, 0:1]
            row = jnp.sum(win * w, axis=0)
            y_ref[pl.ds(rl, 1), :] = row[None, :]
        return carry

    lax.fori_loop(0, _R // 16, decode_body, 0)

    y = y_ref[...]
    h = jnp.maximum(jnp.dot(y, rw0_ref[...],
                            preferred_element_type=jnp.float32) + rb0_ref[...], 0.0)
    h = jnp.maximum(jnp.dot(h, rw1_ref[...],
                            preferred_element_type=jnp.float32) + rb1_ref[...], 0.0)
    o = jnp.dot(h, rw2_ref[...],
                preferred_element_type=jnp.float32) + rb2_ref[...]
    out_ref[0] = o


def kernel(room_positions, rooms, emb, w0, b0, w1, b1, w2, b2,
           rw0, rb0, rw1, rb1, rw2, rb2):
    n = room_positions.shape[0]
    pos_flat = room_positions.astype(jnp.int32).reshape(-1)
    rooms_t = rooms.transpose(0, 2, 3, 1)
    w0t = w0.transpose(2, 3, 1, 0).reshape(5, 5 * _CIN, _C1)
    w1t = w1.transpose(2, 3, 1, 0).reshape(3, 3 * _C1, _C2)
    w2t = w2.transpose(2, 3, 1, 0).reshape(3, 3 * _C2, _C3)
    rw0t, rw1t, rw2t = rw0.T, rw1.T, rw2.T
    b0r, b1r, b2r = b0[None, :], b1[None, :], b2[None, :]
    rb0r, rb1r, rb2r = rb0[None, :], rb1[None, :], rb2[None, :]

    rows = jnp.arange(_BUF, dtype=jnp.int32)
    ry = rows % _YG
    interior = (rows >= _ROW0) & (rows < _ROW0 + _NROW) & (ry >= 2) & (ry < 2 + _MY)
    lane = jnp.arange(_CIN, dtype=jnp.int32)
    bg = (interior[:, None] & (lane[None, :] == 9)).astype(jnp.float32)
    ry_out = jnp.arange(_NROW, dtype=jnp.int32) % _YG
    ym = ((ry_out >= 2) & (ry_out < 2 + _MY)).astype(jnp.float32)[:, None]

    specs = [
        pl.BlockSpec((_R, _WM, _HM, 9), lambda i, p: (0, 0, 0, 0)),
        pl.BlockSpec((_R, _E), lambda i, p: (0, 0)),
        pl.BlockSpec((5, 5 * _CIN, _C1), lambda i, p: (0, 0, 0)),
        pl.BlockSpec((1, _C1), lambda i, p: (0, 0)),
        pl.BlockSpec((3, 3 * _C1, _C2), lambda i, p: (0, 0, 0)),
        pl.BlockSpec((1, _C2), lambda i, p: (0, 0)),
        pl.BlockSpec((3, 3 * _C2, _C3), lambda i, p: (0, 0, 0)),
        pl.BlockSpec((1, _C3), lambda i, p: (0, 0)),
        pl.BlockSpec((_C3, _OUT), lambda i, p: (0, 0)),
        pl.BlockSpec((1, _OUT), lambda i, p: (0, 0)),
        pl.BlockSpec((_OUT, _OUT), lambda i, p: (0, 0)),
        pl.BlockSpec((1, _OUT), lambda i, p: (0, 0)),
        pl.BlockSpec((_OUT, _OUT), lambda i, p: (0, 0)),
        pl.BlockSpec((1, _OUT), lambda i, p: (0, 0)),
        pl.BlockSpec((_BUF, _CIN), lambda i, p: (0, 0)),
        pl.BlockSpec((_NROW, 1), lambda i, p: (0, 0)),
    ]

    grid_spec = pltpu.PrefetchScalarGridSpec(
        num_scalar_prefetch=1,
        grid=(n,),
        in_specs=specs,
        out_specs=pl.BlockSpec((1, _R, _OUT), lambda i, p: (i, 0, 0)),
        scratch_shapes=[
            pltpu.VMEM((_BUF, _CIN), jnp.float32),
            pltpu.VMEM((_BUF, _CIN), jnp.float32),
            pltpu.VMEM((_BUF, _CIN), jnp.float32),
            pltpu.VMEM((_BUF, _CIN), jnp.float32),
            pltpu.VMEM((_NROW + _WIN + 8, _C3), jnp.float32),
            pltpu.VMEM((_R, _C3), jnp.float32),
            pltpu.VMEM((_R, _WIN, _CIN), jnp.float32),
        ],
    )

    return pl.pallas_call(
        _kernel,
        out_shape=jax.ShapeDtypeStruct((n, _R, _OUT), jnp.float32),
        grid_spec=grid_spec,
        compiler_params=pltpu.CompilerParams(
            dimension_semantics=("arbitrary",),
        ),
        name="main_network_fused",
    )(pos_flat, rooms_t, emb, w0t, b0r, w1t, b1r, w2t, b2r,
      rw0t, rb0r, rw1t, rb1r, rw2t, rb2r, bg, ym)


# speedup vs baseline: 1.1749x; 1.0802x over previous
import jax
import jax.numpy as jnp
from jax import lax
from jax.experimental import pallas as pl
from jax.experimental.pallas import tpu as pltpu

_N, _R, _WM, _HM = 512, 64, 6, 6
_E, _MX, _MY = 16, 32, 24
_CIN = 10 + _E
_C1, _C2, _C3 = 32, 64, 64
_OUT = 64

_XG, _YG = 40, 32
_BUF = _XG * _YG
_ROW0 = 4 * _YG
_NROW = _MX * _YG
_WIN = _WM * _YG + 8


def _conv_taps(bf, wg_ref, k, h, cout):
    base = _ROW0 - _YG * h - h
    g = jnp.concatenate([bf[base + ty:base + ty + 1152, :] for ty in range(k)],
                        axis=1)
    acc = jnp.zeros((_NROW, cout), jnp.float32)
    for tx in range(k):
        acc = acc + jnp.dot(g[_YG * tx:_YG * tx + _NROW, :], wg_ref[tx],
                            preferred_element_type=jnp.float32)
    return acc


def _kernel(pos_smem, rooms_ref, emb_ref, w0_ref, b0_ref, w1_ref, b1_ref,
            w2_ref, b2_ref, rw0_ref, rb0_ref, rw1_ref, rb1_ref, rw2_ref,
            rb2_ref, bg_ref, ym_ref, out_ref, m_ref, m2_ref, m3_ref, m4_ref, f_ref, y_ref,
            vals_ref):
    i = pl.program_id(0)

    @pl.when(i == 0)
    def _build():
        p9 = rooms_ref[...]
        mask = p9[:, :, :, 0:1]
        pe = mask * emb_ref[...][:, None, None, :]
        patch = jnp.concatenate(
            [p9, jnp.zeros((_R, _WM, _HM, 1), jnp.float32), pe], axis=-1)
        pw = jnp.concatenate(
            [patch, jnp.zeros((_R, _WM, _YG - _HM, _CIN), jnp.float32)],
            axis=2).reshape(_R, _WM * _YG, _CIN)
        vals_ref[...] = jnp.concatenate(
            [pw, jnp.zeros((_R, 8, _CIN), jnp.float32)], axis=1)

    m_ref[...] = bg_ref[...]
    m2_ref[...] = jnp.zeros((_BUF, _CIN), jnp.float32)
    m3_ref[...] = jnp.zeros((_BUF, _CIN), jnp.float32)
    m4_ref[...] = jnp.zeros((_BUF, _CIN), jnp.float32)
    mrefs = (m_ref, m2_ref, m3_ref, m4_ref)

    def scatter_body(r, carry):
        for l in range(4):
            rl = 4 * r + l
            px = pos_smem[(i * _R + rl) * 2]
            py = pos_smem[(i * _R + rl) * 2 + 1]
            base = (px + 4) * _YG + py + 2
            mr = mrefs[l]
            cur = mr[pl.ds(base, _WM * _YG), :]
            mr[pl.ds(base, _WM * _YG), :] = cur + vals_ref[rl, 0:_WM * _YG, :]
        return carry

    lax.fori_loop(0, _R // 4, scatter_body, 0)

    ym = ym_ref[...]
    bf = (m_ref[...] + m2_ref[...]) + (m3_ref[...] + m4_ref[...])
    a = _conv_taps(bf, w0_ref, 5, 2, _C1)
    a = jnp.maximum(a + b0_ref[...], 0.0) * ym
    bf = jnp.concatenate(
        [jnp.zeros((_ROW0, _C1), jnp.float32), a,
         jnp.zeros((_ROW0, _C1), jnp.float32)], axis=0)
    a = _conv_taps(bf, w1_ref, 3, 1, _C2)
    a = jnp.maximum(a + b1_ref[...], 0.0) * ym
    bf = jnp.concatenate(
        [jnp.zeros((_ROW0, _C2), jnp.float32), a,
         jnp.zeros((_ROW0, _C2), jnp.float32)], axis=0)
    a = _conv_taps(bf, w2_ref, 3, 1, _C3)
    a = jnp.maximum(a + b2_ref[...], 0.0)
    f_ref[0:_NROW, :] = a
    f_ref[_NROW:, :] = jnp.zeros((_WIN + 8, _C3), jnp.float32)

    for rl in range(_R):
        px = pos_smem[(i * _R + rl) * 2]
        py = pos_smem[(i * _R + rl) * 2 + 1]
        base = px * _YG + py + 2
        win = f_ref[pl.ds(base, _WM * _YG), :]
        w = vals_ref[rl, 0:_WM * _YG, 0:1]
        row = jnp.sum(win * w, axis=0)
        y_ref[rl:rl + 1, :] = row[None, :]

    y = y_ref[...]
    h = jnp.maximum(jnp.dot(y, rw0_ref[...],
                            preferred_element_type=jnp.float32) + rb0_ref[...], 0.0)
    h = jnp.maximum(jnp.dot(h, rw1_ref[...],
                            preferred_element_type=jnp.float32) + rb1_ref[...], 0.0)
    o = jnp.dot(h, rw2_ref[...],
                preferred_element_type=jnp.float32) + rb2_ref[...]
    out_ref[0] = o


def kernel(room_positions, rooms, emb, w0, b0, w1, b1, w2, b2,
           rw0, rb0, rw1, rb1, rw2, rb2):
    n = room_positions.shape[0]
    pos_flat = room_positions.astype(jnp.int32).reshape(-1)
    rooms_t = rooms.transpose(0, 2, 3, 1)
    w0t = w0.transpose(2, 3, 1, 0).reshape(5, 5 * _CIN, _C1)
    w1t = w1.transpose(2, 3, 1, 0).reshape(3, 3 * _C1, _C2)
    w2t = w2.transpose(2, 3, 1, 0).reshape(3, 3 * _C2, _C3)
    rw0t, rw1t, rw2t = rw0.T, rw1.T, rw2.T
    b0r, b1r, b2r = b0[None, :], b1[None, :], b2[None, :]
    rb0r, rb1r, rb2r = rb0[None, :], rb1[None, :], rb2[None, :]

    rows = jnp.arange(_BUF, dtype=jnp.int32)
    ry = rows % _YG
    interior = (rows >= _ROW0) & (rows < _ROW0 + _NROW) & (ry >= 2) & (ry < 2 + _MY)
    lane = jnp.arange(_CIN, dtype=jnp.int32)
    bg = (interior[:, None] & (lane[None, :] == 9)).astype(jnp.float32)
    ry_out = jnp.arange(_NROW, dtype=jnp.int32) % _YG
    ym = ((ry_out >= 2) & (ry_out < 2 + _MY)).astype(jnp.float32)[:, None]

    specs = [
        pl.BlockSpec((_R, _WM, _HM, 9), lambda i, p: (0, 0, 0, 0)),
        pl.BlockSpec((_R, _E), lambda i, p: (0, 0)),
        pl.BlockSpec((5, 5 * _CIN, _C1), lambda i, p: (0, 0, 0)),
        pl.BlockSpec((1, _C1), lambda i, p: (0, 0)),
        pl.BlockSpec((3, 3 * _C1, _C2), lambda i, p: (0, 0, 0)),
        pl.BlockSpec((1, _C2), lambda i, p: (0, 0)),
        pl.BlockSpec((3, 3 * _C2, _C3), lambda i, p: (0, 0, 0)),
        pl.BlockSpec((1, _C3), lambda i, p: (0, 0)),
        pl.BlockSpec((_C3, _OUT), lambda i, p: (0, 0)),
        pl.BlockSpec((1, _OUT), lambda i, p: (0, 0)),
        pl.BlockSpec((_OUT, _OUT), lambda i, p: (0, 0)),
        pl.BlockSpec((1, _OUT), lambda i, p: (0, 0)),
        pl.BlockSpec((_OUT, _OUT), lambda i, p: (0, 0)),
        pl.BlockSpec((1, _OUT), lambda i, p: (0, 0)),
        pl.BlockSpec((_BUF, _CIN), lambda i, p: (0, 0)),
        pl.BlockSpec((_NROW, 1), lambda i, p: (0, 0)),
    ]

    grid_spec = pltpu.PrefetchScalarGridSpec(
        num_scalar_prefetch=1,
        grid=(n,),
        in_specs=specs,
        out_specs=pl.BlockSpec((1, _R, _OUT), lambda i, p: (i, 0, 0)),
        scratch_shapes=[
            pltpu.VMEM((_BUF, _CIN), jnp.float32),
            pltpu.VMEM((_BUF, _CIN), jnp.float32),
            pltpu.VMEM((_BUF, _CIN), jnp.float32),
            pltpu.VMEM((_BUF, _CIN), jnp.float32),
            pltpu.VMEM((_NROW + _WIN + 8, _C3), jnp.float32),
            pltpu.VMEM((_R, _C3), jnp.float32),
            pltpu.VMEM((_R, _WIN, _CIN), jnp.float32),
        ],
    )

    return pl.pallas_call(
        _kernel,
        out_shape=jax.ShapeDtypeStruct((n, _R, _OUT), jnp.float32),
        grid_spec=grid_spec,
        compiler_params=pltpu.CompilerParams(
            dimension_semantics=("arbitrary",),
        ),
        name="main_network_fused",
    )(pos_flat, rooms_t, emb, w0t, b0r, w1t, b1r, w2t, b2r,
      rw0t, rb0r, rw1t, rb1r, rw2t, rb2r, bg, ym)


# speedup vs baseline: 1.1992x; 1.0207x over previous
import jax
import jax.numpy as jnp
from jax import lax
from jax.experimental import pallas as pl
from jax.experimental.pallas import tpu as pltpu

_N, _R, _WM, _HM = 512, 64, 6, 6
_E, _MX, _MY = 16, 32, 24
_CIN = 10 + _E
_C1, _C2, _C3 = 32, 64, 64
_OUT = 64

_XG, _YG = 40, 32
_BUF = _XG * _YG
_ROW0 = 4 * _YG
_NROW = _MX * _YG
_WIN = _WM * _YG + 8


def _conv_taps(bf, wg_ref, k, h, cout):
    base = _ROW0 - _YG * h - h
    g = jnp.concatenate([bf[base + ty:base + ty + 1152, :] for ty in range(k)],
                        axis=1)
    acc = jnp.zeros((_NROW, cout), jnp.float32)
    for tx in range(k):
        acc = acc + jnp.dot(g[_YG * tx:_YG * tx + _NROW, :], wg_ref[tx],
                            preferred_element_type=jnp.float32)
    return acc


def _kernel(pos_smem, rooms_ref, emb_ref, w0_ref, b0_ref, w1_ref, b1_ref,
            w2_ref, b2_ref, rw0_ref, rb0_ref, rw1_ref, rb1_ref, rw2_ref,
            rb2_ref, bg_ref, ym_ref, out_ref, m_ref, m2_ref, m3_ref, m4_ref, f_ref, y_ref,
            vals_ref):
    i = pl.program_id(0)

    @pl.when(i == 0)
    def _build():
        p9 = rooms_ref[...]
        mask = p9[:, :, :, 0:1]
        pe = mask * emb_ref[...][:, None, None, :]
        patch = jnp.concatenate(
            [p9, jnp.zeros((_R, _WM, _HM, 1), jnp.float32), pe], axis=-1)
        pw = jnp.concatenate(
            [patch, jnp.zeros((_R, _WM, _YG - _HM, _CIN), jnp.float32)],
            axis=2).reshape(_R, _WM * _YG, _CIN)
        vals_ref[...] = jnp.concatenate(
            [pw, jnp.zeros((_R, 8, _CIN), jnp.float32)], axis=1)

    m_ref[...] = bg_ref[...]
    m2_ref[...] = jnp.zeros((_BUF, _CIN), jnp.float32)
    m3_ref[...] = jnp.zeros((_BUF, _CIN), jnp.float32)
    m4_ref[...] = jnp.zeros((_BUF, _CIN), jnp.float32)
    mrefs = (m_ref, m2_ref, m3_ref, m4_ref)

    for rl in range(_R):
        px = pos_smem[(i * _R + rl) * 2]
        py = pos_smem[(i * _R + rl) * 2 + 1]
        base = (px + 4) * _YG + py + 2
        mr = mrefs[rl % 4]
        cur = mr[pl.ds(base, _WM * _YG), :]
        mr[pl.ds(base, _WM * _YG), :] = cur + vals_ref[rl, 0:_WM * _YG, :]

    ym = ym_ref[...]
    bf = (m_ref[...] + m2_ref[...]) + (m3_ref[...] + m4_ref[...])
    a = _conv_taps(bf, w0_ref, 5, 2, _C1)
    a = jnp.maximum(a + b0_ref[...], 0.0) * ym
    bf = jnp.concatenate(
        [jnp.zeros((_ROW0, _C1), jnp.float32), a,
         jnp.zeros((_ROW0, _C1), jnp.float32)], axis=0)
    a = _conv_taps(bf, w1_ref, 3, 1, _C2)
    a = jnp.maximum(a + b1_ref[...], 0.0) * ym
    bf = jnp.concatenate(
        [jnp.zeros((_ROW0, _C2), jnp.float32), a,
         jnp.zeros((_ROW0, _C2), jnp.float32)], axis=0)
    a = _conv_taps(bf, w2_ref, 3, 1, _C3)
    a = jnp.maximum(a + b2_ref[...], 0.0)
    f_ref[0:_NROW, :] = a
    f_ref[_NROW:, :] = jnp.zeros((_WIN + 8, _C3), jnp.float32)

    for rl in range(_R):
        px = pos_smem[(i * _R + rl) * 2]
        py = pos_smem[(i * _R + rl) * 2 + 1]
        base = px * _YG + py + 2
        win = f_ref[pl.ds(base, _WM * _YG), :]
        w = vals_ref[rl, 0:_WM * _YG, 0:1]
        row = jnp.sum(win * w, axis=0)
        y_ref[rl:rl + 1, :] = row[None, :]

    y = y_ref[...]
    h = jnp.maximum(jnp.dot(y, rw0_ref[...],
                            preferred_element_type=jnp.float32) + rb0_ref[...], 0.0)
    h = jnp.maximum(jnp.dot(h, rw1_ref[...],
                            preferred_element_type=jnp.float32) + rb1_ref[...], 0.0)
    o = jnp.dot(h, rw2_ref[...],
                preferred_element_type=jnp.float32) + rb2_ref[...]
    out_ref[0] = o


def kernel(room_positions, rooms, emb, w0, b0, w1, b1, w2, b2,
           rw0, rb0, rw1, rb1, rw2, rb2):
    n = room_positions.shape[0]
    pos_flat = room_positions.astype(jnp.int32).reshape(-1)
    rooms_t = rooms.transpose(0, 2, 3, 1)
    w0t = w0.transpose(2, 3, 1, 0).reshape(5, 5 * _CIN, _C1)
    w1t = w1.transpose(2, 3, 1, 0).reshape(3, 3 * _C1, _C2)
    w2t = w2.transpose(2, 3, 1, 0).reshape(3, 3 * _C2, _C3)
    rw0t, rw1t, rw2t = rw0.T, rw1.T, rw2.T
    b0r, b1r, b2r = b0[None, :], b1[None, :], b2[None, :]
    rb0r, rb1r, rb2r = rb0[None, :], rb1[None, :], rb2[None, :]

    rows = jnp.arange(_BUF, dtype=jnp.int32)
    ry = rows % _YG
    interior = (rows >= _ROW0) & (rows < _ROW0 + _NROW) & (ry >= 2) & (ry < 2 + _MY)
    lane = jnp.arange(_CIN, dtype=jnp.int32)
    bg = (interior[:, None] & (lane[None, :] == 9)).astype(jnp.float32)
    ry_out = jnp.arange(_NROW, dtype=jnp.int32) % _YG
    ym = ((ry_out >= 2) & (ry_out < 2 + _MY)).astype(jnp.float32)[:, None]

    specs = [
        pl.BlockSpec((_R, _WM, _HM, 9), lambda i, p: (0, 0, 0, 0)),
        pl.BlockSpec((_R, _E), lambda i, p: (0, 0)),
        pl.BlockSpec((5, 5 * _CIN, _C1), lambda i, p: (0, 0, 0)),
        pl.BlockSpec((1, _C1), lambda i, p: (0, 0)),
        pl.BlockSpec((3, 3 * _C1, _C2), lambda i, p: (0, 0, 0)),
        pl.BlockSpec((1, _C2), lambda i, p: (0, 0)),
        pl.BlockSpec((3, 3 * _C2, _C3), lambda i, p: (0, 0, 0)),
        pl.BlockSpec((1, _C3), lambda i, p: (0, 0)),
        pl.BlockSpec((_C3, _OUT), lambda i, p: (0, 0)),
        pl.BlockSpec((1, _OUT), lambda i, p: (0, 0)),
        pl.BlockSpec((_OUT, _OUT), lambda i, p: (0, 0)),
        pl.BlockSpec((1, _OUT), lambda i, p: (0, 0)),
        pl.BlockSpec((_OUT, _OUT), lambda i, p: (0, 0)),
        pl.BlockSpec((1, _OUT), lambda i, p: (0, 0)),
        pl.BlockSpec((_BUF, _CIN), lambda i, p: (0, 0)),
        pl.BlockSpec((_NROW, 1), lambda i, p: (0, 0)),
    ]

    grid_spec = pltpu.PrefetchScalarGridSpec(
        num_scalar_prefetch=1,
        grid=(n,),
        in_specs=specs,
        out_specs=pl.BlockSpec((1, _R, _OUT), lambda i, p: (i, 0, 0)),
        scratch_shapes=[
            pltpu.VMEM((_BUF, _CIN), jnp.float32),
            pltpu.VMEM((_BUF, _CIN), jnp.float32),
            pltpu.VMEM((_BUF, _CIN), jnp.float32),
            pltpu.VMEM((_BUF, _CIN), jnp.float32),
            pltpu.VMEM((_NROW + _WIN + 8, _C3), jnp.float32),
            pltpu.VMEM((_R, _C3), jnp.float32),
            pltpu.VMEM((_R, _WIN, _CIN), jnp.float32),
        ],
    )

    return pl.pallas_call(
        _kernel,
        out_shape=jax.ShapeDtypeStruct((n, _R, _OUT), jnp.float32),
        grid_spec=grid_spec,
        compiler_params=pltpu.CompilerParams(
            dimension_semantics=("arbitrary",),
        ),
        name="main_network_fused",
    )(pos_flat, rooms_t, emb, w0t, b0r, w1t, b1r, w2t, b2r,
      rw0t, rb0r, rw1t, rb1r, rw2t, rb2r, bg, ym)
